# baseline (speedup 1.0000x reference)
import jax
import jax.numpy as jnp
from jax import lax
from jax.experimental import pallas as pl
from jax.experimental.pallas import tpu as pltpu
from jax.experimental.pallas import tpu_sc as plsc

N_FIELDS = 26
BATCH = 4096
VOCAB = 100000
EMB = 64
COL_ENC = 32
_OUTW = EMB + COL_ENC

_NC = 2
_NS = 16
_NW = _NC * _NS
_BPW = BATCH // _NW
_CREP = 32


def _body(gidx_hbm, tables_hbm, colenc_hbm, out_hbm, idx_all, rows, cenc, sem):
    wid = lax.axis_index("s") * _NC + lax.axis_index("c")
    b0 = wid * _BPW

    pltpu.sync_copy(gidx_hbm.at[:, pl.ds(b0, _BPW)], idx_all)

    for r in range(_CREP):
        pltpu.sync_copy(colenc_hbm, cenc.at[r])

    cps = [pltpu.async_copy(tables_hbm.at[idx_all.at[0]], rows.at[0], sem),
           None]
    for f in range(N_FIELDS):
        if f + 1 < N_FIELDS:
            cps[(f + 1) % 2] = pltpu.async_copy(
                tables_hbm.at[idx_all.at[f + 1]], rows.at[(f + 1) % 2], sem)
        cps[f % 2].wait()
        pltpu.sync_copy(rows.at[f % 2],
                        out_hbm.at[pl.ds(b0, _BPW), f, pl.ds(0, EMB)])

    for c in range(_BPW // _CREP):
        pltpu.sync_copy(cenc, out_hbm.at[pl.ds(b0 + c * _CREP, _CREP), slice(None),
                                         pl.ds(EMB, COL_ENC)])


@jax.jit
def _call(gidx, tables_flat, col_encoding):
    mesh = plsc.VectorSubcoreMesh(core_axis_name="c", subcore_axis_name="s")
    run = pl.kernel(
        _body,
        out_type=jax.ShapeDtypeStruct((BATCH, N_FIELDS, _OUTW), jnp.float32),
        mesh=mesh,
        scratch_types=[
            pltpu.VMEM((N_FIELDS, _BPW), jnp.int32),
            pltpu.VMEM((2, _BPW, EMB), jnp.float32),
            pltpu.VMEM((_CREP, N_FIELDS, COL_ENC), jnp.float32),
            pltpu.SemaphoreType.DMA,
        ],
        compiler_params=pltpu.CompilerParams(use_tc_tiling_on_sc=False),
    )
    return run(gidx, tables_flat, col_encoding)


def kernel(indices, tables, col_encoding):
    gidx = indices.astype(jnp.int32) + (
        jnp.arange(N_FIELDS, dtype=jnp.int32) * VOCAB)[:, None]
    tables_flat = tables.reshape(N_FIELDS * VOCAB, EMB)
    return _call(gidx, tables_flat, col_encoding)


# speedup vs baseline: 1.0061x; 1.0061x over previous
import jax
import jax.numpy as jnp
from jax import lax
from jax.experimental import pallas as pl
from jax.experimental.pallas import tpu as pltpu
from jax.experimental.pallas import tpu_sc as plsc

N_FIELDS = 26
BATCH = 4096
VOCAB = 100000
EMB = 64
COL_ENC = 32
_OUTW = EMB + COL_ENC

_NC = 2
_NS = 16
_NW = _NC * _NS
_BPW = BATCH // _NW
_CREP = 32
_NBUF = 8


def _body(gidx_hbm, tables_hbm, colenc_hbm, out_hbm, idx_all, rows, cenc,
          gsem, csem, osem):
    wid = lax.axis_index("s") * _NC + lax.axis_index("c")
    b0 = wid * _BPW

    ccps = [pltpu.async_copy(colenc_hbm, cenc.at[r], csem)
            for r in range(_CREP)]

    pltpu.sync_copy(gidx_hbm.at[:, pl.ds(b0, _BPW)], idx_all)

    gcps = [None] * N_FIELDS
    ocps = [None] * N_FIELDS
    for f in range(_NBUF):
        gcps[f] = pltpu.async_copy(tables_hbm.at[idx_all.at[f]],
                                   rows.at[f], gsem)
    for f in range(N_FIELDS):
        gcps[f].wait()
        ocps[f] = pltpu.async_copy(
            rows.at[f % _NBUF],
            out_hbm.at[pl.ds(b0, _BPW), f, pl.ds(0, EMB)], osem)
        nf = f + _NBUF
        if nf < N_FIELDS:
            ocps[nf - _NBUF].wait()
            gcps[nf] = pltpu.async_copy(tables_hbm.at[idx_all.at[nf]],
                                        rows.at[nf % _NBUF], gsem)
    for f in range(N_FIELDS - _NBUF, N_FIELDS):
        ocps[f].wait()

    for cp in ccps:
        cp.wait()
    wcps = [pltpu.async_copy(
        cenc, out_hbm.at[pl.ds(b0 + c * _CREP, _CREP), slice(None),
                         pl.ds(EMB, COL_ENC)], osem)
        for c in range(_BPW // _CREP)]
    for cp in wcps:
        cp.wait()


@jax.jit
def _call(gidx, tables_flat, col_encoding):
    mesh = plsc.VectorSubcoreMesh(core_axis_name="c", subcore_axis_name="s")
    run = pl.kernel(
        _body,
        out_type=jax.ShapeDtypeStruct((BATCH, N_FIELDS, _OUTW), jnp.float32),
        mesh=mesh,
        scratch_types=[
            pltpu.VMEM((N_FIELDS, _BPW), jnp.int32),
            pltpu.VMEM((_NBUF, _BPW, EMB), jnp.float32),
            pltpu.VMEM((_CREP, N_FIELDS, COL_ENC), jnp.float32),
            pltpu.SemaphoreType.DMA,
            pltpu.SemaphoreType.DMA,
            pltpu.SemaphoreType.DMA,
        ],
        compiler_params=pltpu.CompilerParams(use_tc_tiling_on_sc=False),
    )
    return run(gidx, tables_flat, col_encoding)


def kernel(indices, tables, col_encoding):
    gidx = indices.astype(jnp.int32) + (
        jnp.arange(N_FIELDS, dtype=jnp.int32) * VOCAB)[:, None]
    tables_flat = tables.reshape(N_FIELDS * VOCAB, EMB)
    return _call(gidx, tables_flat, col_encoding)


# speedup vs baseline: 1.4319x; 1.4232x over previous
import jax
import jax.numpy as jnp
from jax import lax
from jax.experimental import pallas as pl
from jax.experimental.pallas import tpu as pltpu
from jax.experimental.pallas import tpu_sc as plsc

N_FIELDS = 26
BATCH = 4096
VOCAB = 100000
EMB = 64
COL_ENC = 32
_OUTW = EMB + COL_ENC

_NC = 2
_NS = 16
_NW = _NC * _NS
_BPW = BATCH // _NW
_CH = 16
_NCHUNK = _BPW // _CH
_NSLOT = 3


def _body(gidx_hbm, tab_hbm, cenc_hbm, out_hbm, idxv, gbuf, slab, cencv,
          sem0, sem1, sem2):
    sems = [sem0, sem1, sem2]
    wid = lax.axis_index("s") * _NC + lax.axis_index("c")
    b0 = wid * _BPW
    iota = lax.iota(jnp.int32, 16)

    pltpu.sync_copy(gidx_hbm.at[:, pl.ds(b0, _BPW)], idxv)
    pltpu.sync_copy(cenc_hbm, cencv)

    for f in range(N_FIELDS):
        for k2 in range(COL_ENC // 16):
            vec = cencv[f, pl.ds(16 * k2, 16)]
            for r in range(_CH):
                slab[r, f, pl.ds(EMB + 16 * k2, 16)] = vec

    def _issue(c, f, k):
        vrow = idxv[f, pl.ds(c * _CH, _CH)]
        for j in range(_CH):
            v8 = pl.multiple_of((vrow[j] >> 3) << 3, 8)
            pltpu.async_copy(tab_hbm.at[f, pl.ds(v8, 8), :],
                             gbuf.at[k, pl.ds(8 * j, 8), :], sems[k])

    def _drain(k):
        for j in range(_CH):
            pltpu.make_async_copy(tab_hbm.at[0, pl.ds(0, 8), :],
                                  gbuf.at[k, pl.ds(8 * j, 8), :],
                                  sems[k]).wait()

    def _extract(c, f, k):
        rows = (idxv[f, pl.ds(c * _CH, _CH)] & 7) + iota * 8
        for j in range(_CH):
            row = rows[j]
            for kk in range(EMB // 16):
                vec = gbuf[k, row, pl.ds(16 * kk, 16)]
                slab[j, f, pl.ds(16 * kk, 16)] = vec

    def _chunk(c, carry):
        for k in range(_NSLOT):
            _issue(c, k, k)

        def _inner(i, icarry):
            for k in range(_NSLOT):
                f_old = _NSLOT * (i - 1) + k
                f_new = _NSLOT * i + k
                _drain(k)
                _extract(c, f_old, k)

                @pl.when(f_new < N_FIELDS)
                def _():
                    _issue(c, f_new, k)
            return icarry

        lax.fori_loop(1, 9, _inner, 0)
        for k in range(2):
            _drain(k)
            _extract(c, N_FIELDS - 2 + k, k)
        pltpu.sync_copy(slab, out_hbm.at[pl.ds(b0 + c * _CH, _CH), :, :])
        return carry

    lax.fori_loop(0, _NCHUNK, _chunk, 0)


@jax.jit
def _call(gidx, tables, col_encoding):
    mesh = plsc.VectorSubcoreMesh(core_axis_name="c", subcore_axis_name="s")
    run = pl.kernel(
        _body,
        out_type=jax.ShapeDtypeStruct((BATCH, N_FIELDS, _OUTW), jnp.float32),
        mesh=mesh,
        scratch_types=[
            pltpu.VMEM((N_FIELDS, _BPW), jnp.int32),
            pltpu.VMEM((_NSLOT, 8 * _CH, EMB), jnp.float32),
            pltpu.VMEM((_CH, N_FIELDS, _OUTW), jnp.float32),
            pltpu.VMEM((N_FIELDS, COL_ENC), jnp.float32),
            pltpu.SemaphoreType.DMA,
            pltpu.SemaphoreType.DMA,
            pltpu.SemaphoreType.DMA,
        ],
    )
    return run(gidx, tables, col_encoding)


def kernel(indices, tables, col_encoding):
    return _call(indices.astype(jnp.int32), tables, col_encoding)


# speedup vs baseline: 1.5822x; 1.1050x over previous
import jax
import jax.numpy as jnp
from jax import lax
from jax.experimental import pallas as pl
from jax.experimental.pallas import tpu as pltpu
from jax.experimental.pallas import tpu_sc as plsc

N_FIELDS = 26
BATCH = 4096
VOCAB = 100000
EMB = 64
COL_ENC = 32
_OUTW = EMB + COL_ENC

_NC = 2
_NS = 16
_NW = _NC * _NS
_BPW = BATCH // _NW
_CH = 16
_NCHUNK = _BPW // _CH
_NSLOT = 3


def _body(gidx_hbm, tab_hbm, cenc_hbm, out_hbm, idxv, gbuf, slab, cencv,
          sem0, sem1, sem2):
    sems = [sem0, sem1, sem2]
    wid = lax.axis_index("s") * _NC + lax.axis_index("c")
    b0 = wid * _BPW
    iota = lax.iota(jnp.int32, 16)

    pltpu.sync_copy(gidx_hbm.at[:, pl.ds(b0, _BPW)], idxv)
    pltpu.sync_copy(cenc_hbm, cencv)

    for f in range(N_FIELDS):
        for k2 in range(COL_ENC // 16):
            vec = cencv[f, pl.ds(16 * k2, 16)]
            for r in range(_CH):
                slab[r, f, pl.ds(EMB + 16 * k2, 16)] = vec

    def _issue(c, f, k):
        vrow = idxv[f, pl.ds(c * _CH, _CH)]
        for j in range(_CH):
            pltpu.async_copy(tab_hbm.at[f, pl.ds(vrow[j], 1), :],
                             gbuf.at[k, pl.ds(j, 1), :], sems[k])

    def _drain(k):
        for j in range(_CH):
            pltpu.make_async_copy(tab_hbm.at[0, pl.ds(0, 1), :],
                                  gbuf.at[k, pl.ds(j, 1), :],
                                  sems[k]).wait()

    def _extract(c, f, k):
        for j in range(_CH):
            for kk in range(EMB // 16):
                vec = gbuf[k, j, pl.ds(16 * kk, 16)]
                slab[j, f, pl.ds(16 * kk, 16)] = vec

    def _chunk(c, carry):
        for k in range(_NSLOT):
            _issue(c, k, k)

        def _inner(i, icarry):
            for k in range(_NSLOT):
                f_old = _NSLOT * (i - 1) + k
                f_new = _NSLOT * i + k
                _drain(k)
                _extract(c, f_old, k)

                @pl.when(f_new < N_FIELDS)
                def _():
                    _issue(c, f_new, k)
            return icarry

        lax.fori_loop(1, 9, _inner, 0)
        for k in range(2):
            _drain(k)
            _extract(c, N_FIELDS - 2 + k, k)
        pltpu.sync_copy(slab, out_hbm.at[pl.ds(b0 + c * _CH, _CH), :, :])
        return carry

    lax.fori_loop(0, _NCHUNK, _chunk, 0)


@jax.jit
def _call(gidx, tables, col_encoding):
    mesh = plsc.VectorSubcoreMesh(core_axis_name="c", subcore_axis_name="s")
    run = pl.kernel(
        _body,
        out_type=jax.ShapeDtypeStruct((BATCH, N_FIELDS, _OUTW), jnp.float32),
        mesh=mesh,
        scratch_types=[
            pltpu.VMEM((N_FIELDS, _BPW), jnp.int32),
            pltpu.VMEM((_NSLOT, _CH, EMB), jnp.float32),
            pltpu.VMEM((_CH, N_FIELDS, _OUTW), jnp.float32),
            pltpu.VMEM((N_FIELDS, COL_ENC), jnp.float32),
            pltpu.SemaphoreType.DMA,
            pltpu.SemaphoreType.DMA,
            pltpu.SemaphoreType.DMA,
        ],
    )
    return run(gidx, tables, col_encoding)


def kernel(indices, tables, col_encoding):
    return _call(indices.astype(jnp.int32), tables, col_encoding)
